# Initial kernel scaffold; baseline (speedup 1.0000x reference)
import jax
import jax.numpy as jnp
from jax.experimental import pallas as pl


def kernel(x, edge_index, edge_attr):
    raise NotImplementedError("write your pallas kernel here")


# speedup vs baseline: 4.0242x; 4.0242x over previous
import functools

import jax
import jax.numpy as jnp
from jax import lax
from jax.experimental import pallas as pl
from jax.experimental.pallas import tpu as pltpu
from jax.experimental.pallas import tpu_sc as plsc

GAMMA_CONST = 12.0
EPS = 1e-12


def _normalize_body(x_ref, o_ref):
    v = x_ref[...]
    n = jnp.sqrt(jnp.sum(v * v, axis=1, keepdims=True))
    o_ref[...] = v / jnp.maximum(n, EPS)


def _normalize_rows(x):
    n_nodes, d = x.shape
    return pl.pallas_call(
        _normalize_body,
        out_shape=jax.ShapeDtypeStruct((n_nodes, d), jnp.float32),
    )(x)


def _make_sc_scorer(n_nodes, d, e_total):
    info = plsc.get_sparse_core_info()
    nc, ns, lanes = info.num_cores, info.num_subcores, info.num_lanes
    nw = nc * ns
    assert e_total % nw == 0
    epw = e_total // nw
    ch = 80
    assert epw % ch == 0
    n_chunks = epw // ch
    n_seg = d // lanes

    mesh = plsc.VectorSubcoreMesh(core_axis_name="c", subcore_axis_name="s")

    @functools.partial(
        pl.kernel,
        mesh=mesh,
        compiler_params=pltpu.CompilerParams(needs_layout_passes=False),
        out_type=jax.ShapeDtypeStruct((e_total,), jnp.float32),
        scratch_types=[
            pltpu.VMEM((epw,), jnp.int32),
            pltpu.VMEM((epw,), jnp.int32),
            pltpu.VMEM((ch, d), jnp.float32),
            pltpu.VMEM((ch, d), jnp.float32),
            pltpu.VMEM((ch, 2 * d), jnp.float32),
            pltpu.VMEM((epw,), jnp.float32),
            pltpu.SemaphoreType.DMA,
            pltpu.SemaphoreType.DMA,
            pltpu.SemaphoreType.DMA,
        ],
    )
    def scorer(xn_hbm, src_hbm, dst_hbm, rel_hbm, out_hbm,
               src_v, dst_v, head_v, tail_v, rel_v, out_v,
               sem_h, sem_t, sem_r):
        wid = lax.axis_index("s") * nc + lax.axis_index("c")
        base = wid * epw
        pltpu.sync_copy(src_hbm.at[pl.ds(base, epw)], src_v)
        pltpu.sync_copy(dst_hbm.at[pl.ds(base, epw)], dst_v)

        def chunk_body(k, _):
            off = k * ch
            cp_h = pltpu.async_copy(
                xn_hbm.at[src_v.at[pl.ds(off, ch)]], head_v, sem_h)
            cp_t = pltpu.async_copy(
                xn_hbm.at[dst_v.at[pl.ds(off, ch)]], tail_v, sem_t)
            cp_r = pltpu.async_copy(
                rel_hbm.at[pl.ds(base + off, ch)], rel_v, sem_r)
            cp_h.wait()
            cp_t.wait()
            cp_r.wait()

            lane_ids = lax.iota(jnp.int32, 16)

            def group_body(g, _):
                def edge_body(e16, res):
                    e = g * lanes + e16
                    acc = jnp.zeros((lanes,), jnp.float32)
                    for j in range(n_seg):
                        h = head_v[e, pl.ds(j * lanes, lanes)]
                        t = tail_v[e, pl.ds(j * lanes, lanes)]
                        ra = rel_v[e, pl.ds(j * lanes, lanes)]
                        rb = rel_v[e, pl.ds(d + j * lanes, lanes)]
                        acc = acc + jnp.abs(h * ra - t * rb)
                    s = GAMMA_CONST - jnp.sum(acc)
                    return jnp.where(lane_ids == e16, s, res)

                res = lax.fori_loop(0, lanes, edge_body,
                                    jnp.zeros((lanes,), jnp.float32))
                out_v[pl.ds(off + g * lanes, lanes)] = res
                return 0

            lax.fori_loop(0, ch // lanes, group_body, 0)
            return 0

        lax.fori_loop(0, n_chunks, chunk_body, 0)
        pltpu.sync_copy(out_v, out_hbm.at[pl.ds(base, epw)])

    return scorer


def kernel(x, edge_index, edge_attr):
    n_nodes, d = x.shape
    e_total = edge_attr.shape[0]
    xn = _normalize_rows(x)
    src = edge_index[0].astype(jnp.int32)
    dst = edge_index[1].astype(jnp.int32)
    scorer = _make_sc_scorer(n_nodes, d, e_total)
    score = scorer(xn, src, dst, edge_attr)
    return score.reshape(e_total, 1)


# speedup vs baseline: 7.1600x; 1.7792x over previous
import functools

import jax
import jax.numpy as jnp
from jax import lax
from jax.experimental import pallas as pl
from jax.experimental.pallas import tpu as pltpu
from jax.experimental.pallas import tpu_sc as plsc

GAMMA_CONST = 12.0
EPS = 1e-12


def _normalize_body(x_ref, o_ref):
    v = x_ref[...]
    n = jnp.sqrt(jnp.sum(v * v, axis=1, keepdims=True))
    o_ref[...] = v / jnp.maximum(n, EPS)


def _normalize_rows(x):
    n_nodes, d = x.shape
    return pl.pallas_call(
        _normalize_body,
        out_shape=jax.ShapeDtypeStruct((n_nodes, d), jnp.float32),
    )(x)


def _make_sc_scorer(n_nodes, d, e_total):
    info = plsc.get_sparse_core_info()
    nc, ns, lanes = info.num_cores, info.num_subcores, info.num_lanes
    nw = nc * ns
    assert e_total % nw == 0
    epw = e_total // nw
    ch = 80
    assert epw % ch == 0
    n_chunks = epw // ch
    n_seg = d // lanes

    mesh = plsc.VectorSubcoreMesh(core_axis_name="c", subcore_axis_name="s")

    buf_types = [
        pltpu.VMEM((ch, d), jnp.float32),
        pltpu.VMEM((ch, d), jnp.float32),
        pltpu.VMEM((ch, 2 * d), jnp.float32),
        pltpu.SemaphoreType.DMA,
        pltpu.SemaphoreType.DMA,
        pltpu.SemaphoreType.DMA,
    ]

    @functools.partial(
        pl.kernel,
        mesh=mesh,
        compiler_params=pltpu.CompilerParams(needs_layout_passes=False),
        out_type=jax.ShapeDtypeStruct((e_total,), jnp.float32),
        scratch_types=[
            pltpu.VMEM((epw,), jnp.int32),
            pltpu.VMEM((epw,), jnp.int32),
            pltpu.VMEM((epw,), jnp.float32),
        ] + buf_types + buf_types,
    )
    def scorer(xn_hbm, src_hbm, dst_hbm, rel_hbm, out_hbm,
               src_v, dst_v, out_v,
               head0, tail0, rel0, sh0, st0, sr0,
               head1, tail1, rel1, sh1, st1, sr1):
        wid = lax.axis_index("s") * nc + lax.axis_index("c")
        base = wid * epw
        pltpu.sync_copy(src_hbm.at[pl.ds(base, epw)], src_v)
        pltpu.sync_copy(dst_hbm.at[pl.ds(base, epw)], dst_v)
        bufs = ((head0, tail0, rel0, sh0, st0, sr0),
                (head1, tail1, rel1, sh1, st1, sr1))

        def copies(off, buf):
            head_b, tail_b, rel_b, sh, st, sr = buf
            return (
                pltpu.make_async_copy(
                    xn_hbm.at[src_v.at[pl.ds(off, ch)]], head_b, sh),
                pltpu.make_async_copy(
                    xn_hbm.at[dst_v.at[pl.ds(off, ch)]], tail_b, st),
                pltpu.make_async_copy(
                    rel_hbm.at[pl.ds(base + off, ch)], rel_b, sr),
            )

        def issue(off, buf):
            for cp in copies(off, buf):
                cp.start()

        def compute(off, buf):
            head_b, tail_b, rel_b, _, _, _ = buf
            lane_ids = lax.iota(jnp.int32, 16)

            def group_body(g, _):
                def edge_body(e16, res):
                    e = g * lanes + e16
                    acc = jnp.zeros((lanes,), jnp.float32)
                    for j in range(n_seg):
                        h = head_b[e, pl.ds(j * lanes, lanes)]
                        t = tail_b[e, pl.ds(j * lanes, lanes)]
                        ra = rel_b[e, pl.ds(j * lanes, lanes)]
                        rb = rel_b[e, pl.ds(d + j * lanes, lanes)]
                        acc = acc + jnp.abs(h * ra - t * rb)
                    s = GAMMA_CONST - jnp.sum(acc)
                    return jnp.where(lane_ids == e16, s, res)

                res = lax.fori_loop(0, lanes, edge_body,
                                    jnp.zeros((lanes,), jnp.float32))
                out_v[pl.ds(off + g * lanes, lanes)] = res
                return 0

            lax.fori_loop(0, ch // lanes, group_body, 0)

        def drain(off, buf):
            for cp in copies(off, buf):
                cp.wait()

        issue(0, bufs[0])

        def pair_body(i, _):
            off0 = (2 * i) * ch
            issue(off0 + ch, bufs[1])
            drain(off0, bufs[0])
            compute(off0, bufs[0])
            issue(off0 + 2 * ch, bufs[0])
            drain(off0 + ch, bufs[1])
            compute(off0 + ch, bufs[1])
            return 0

        lax.fori_loop(0, (n_chunks - 1) // 2, pair_body, 0)
        last = (n_chunks - 1) * ch
        drain(last, bufs[0])
        compute(last, bufs[0])
        pltpu.sync_copy(out_v, out_hbm.at[pl.ds(base, epw)])

    return scorer


def kernel(x, edge_index, edge_attr):
    n_nodes, d = x.shape
    e_total = edge_attr.shape[0]
    xn = _normalize_rows(x)
    src = edge_index[0].astype(jnp.int32)
    dst = edge_index[1].astype(jnp.int32)
    scorer = _make_sc_scorer(n_nodes, d, e_total)
    score = scorer(xn, src, dst, edge_attr)
    return score.reshape(e_total, 1)
